# baseline (speedup 1.0000x reference)
import math

import jax
import jax.numpy as jnp
from jax import lax
from jax.experimental import pallas as pl
from jax.experimental.pallas import tpu as pltpu
from jax.experimental.pallas import tpu_sc as plsc

import functools


def _make_sc_lookup(V, D, FLAT, L):
    info = plsc.get_sparse_core_info()
    NC, NS, NL = info.num_cores, info.num_subcores, info.num_lanes
    NW = NC * NS

    C = 128
    assert FLAT % (NW * C) == 0
    PER_W = FLAT // NW
    NCHUNK = PER_W // C
    assert NCHUNK % 2 == 0 and NCHUNK >= 4
    assert PER_W % L == 0
    assert D == 4 * NL and V % 2 == 0 and L % 2 == 0
    scale2 = float(D)
    C2 = C // 2
    DV = D // NL

    mesh = plsc.VectorSubcoreMesh(core_axis_name="c", subcore_axis_name="s")

    @functools.partial(
        pl.kernel,
        mesh=mesh,
        compiler_params=pltpu.CompilerParams(use_tc_tiling_on_sc=True, needs_layout_passes=False),
        out_type=jax.ShapeDtypeStruct((FLAT // 2, 2 * D), jnp.float32),
        scratch_types=[
            pltpu.VMEM((C,), jnp.int32),
            pltpu.VMEM((C,), jnp.int32),
            pltpu.VMEM((C,), jnp.int32),
            pltpu.VMEM((C,), jnp.int32),
            pltpu.VMEM((2, C, 2 * D), jnp.float32),
            pltpu.VMEM((2, C2, 2 * D), jnp.float32),
            pltpu.VMEM((L, 2 * D), jnp.float32),
            pltpu.SemaphoreType.DMA,
            pltpu.SemaphoreType.DMA,
            pltpu.SemaphoreType.DMA,
            pltpu.SemaphoreType.DMA,
            pltpu.SemaphoreType.DMA,
            pltpu.SemaphoreType.DMA,
        ],
    )
    def lookup(table_hbm, idx_hbm, pe_hbm, out_hbm, x0, x1, h0, h1, rin_v,
               rout_v, pe_v, g0, g1, i0, i1, s0, s1):
        idxs = (x0, x1)
        halfs = (h0, h1)
        gsem = (g0, g1)
        isem = (i0, i1)
        ssem = (s0, s1)
        wid = lax.axis_index("s") * NC + lax.axis_index("c")
        gbase = wid * NCHUNK
        obase = wid * (PER_W // 2)

        lanes = lax.broadcasted_iota(jnp.int32, (NL,), 0)

        def halve(src, dst):
            for k in range(C // NL):
                s_ = pl.ds(k * NL, NL)
                dst[s_] = lax.shift_right_logical(src[s_], 1)

        pltpu.sync_copy(pe_hbm, pe_v)
        pltpu.sync_copy(idx_hbm.at[gbase], idxs[0])
        pltpu.async_copy(idx_hbm.at[gbase + 1], idxs[1], isem[1])
        halve(idxs[0], halfs[0])
        pltpu.async_copy(table_hbm.at[halfs[0]], rin_v.at[0], gsem[0])

        @pl.loop(0, NCHUNK, step=2)
        def chunks(cc):
            for b in range(2):
                c = cc + b
                pltpu.make_async_copy(
                    table_hbm.at[halfs[b]], rin_v.at[b], gsem[b]).wait()

                @pl.when(c + 1 < NCHUNK)
                def _():
                    pltpu.make_async_copy(
                        idx_hbm.at[gbase + c + 1], idxs[1 - b],
                        isem[1 - b]).wait()
                    halve(idxs[1 - b], halfs[1 - b])
                    pltpu.async_copy(
                        table_hbm.at[halfs[1 - b]], rin_v.at[1 - b],
                        gsem[1 - b])

                @pl.when(c >= 2)
                def _():
                    pltpu.make_async_copy(
                        rout_v.at[b],
                        out_hbm.at[pl.ds(obase + (c - 2) * C2, C2)],
                        ssem[b]).wait()

                rin = rin_v.at[b]
                rout = rout_v.at[b]
                xi = idxs[b]
                pp = lax.rem(c * C2, L // 2)

                @plsc.parallel_loop(0, C2, unroll=2)
                def row(r2):
                    pr = pp + r2
                    for h in range(2):
                        r = 2 * r2 + h
                        rowv = jnp.zeros((NL,), jnp.int32) + r
                        raw = plsc.load_gather(xi, [rowv])
                        colbase = (raw & 1) * D + lanes
                        for d in range(DV):
                            val = plsc.load_gather(
                                rin, [rowv, colbase + d * NL])
                            s_ = pl.ds(h * D + d * NL, NL)
                            rout[r2, s_] = val * scale2 + pe_v[pr, s_]

                @pl.when(c + 2 < NCHUNK)
                def _():
                    pltpu.async_copy(
                        idx_hbm.at[gbase + c + 2], idxs[b], isem[b])

                pltpu.async_copy(rout_v.at[b],
                                 out_hbm.at[pl.ds(obase + c * C2, C2)],
                                 ssem[b])

        for b in range(2):
            c = NCHUNK - 2 + b
            pltpu.make_async_copy(
                rout_v.at[b], out_hbm.at[pl.ds(obase + c * C2, C2)],
                ssem[b]).wait()

    return lookup


def kernel(x, table, pe):
    B, L = x.shape
    V, D = table.shape
    FLAT = B * L
    x_rows = x.reshape(FLAT // 128, 128).astype(jnp.int32)
    table_pairs = table.reshape(V // 2, 2 * D)
    pe2 = jnp.tile(pe[:L] * math.sqrt(D), (2, 1)).reshape(L, 2 * D)
    out = _make_sc_lookup(V, D, FLAT, L)(table_pairs, x_rows, pe2)
    return out.reshape(B, L, D)


# speedup vs baseline: 1.0633x; 1.0633x over previous
import math

import jax
import jax.numpy as jnp
from jax import lax
from jax.experimental import pallas as pl
from jax.experimental.pallas import tpu as pltpu
from jax.experimental.pallas import tpu_sc as plsc

import functools


def _make_sc_lookup(V, D, FLAT, L):
    info = plsc.get_sparse_core_info()
    NC, NS, NL = info.num_cores, info.num_subcores, info.num_lanes
    NW = NC * NS

    C = 128
    assert FLAT % (NW * C) == 0
    PER_W = FLAT // NW
    NCHUNK = PER_W // C
    assert NCHUNK % 2 == 0 and NCHUNK >= 4
    assert PER_W % L == 0
    assert D % NL == 0 and 2 * D == 128
    scale2 = float(D)
    C2 = C // 2

    mesh = plsc.VectorSubcoreMesh(core_axis_name="c", subcore_axis_name="s")

    @functools.partial(
        pl.kernel,
        mesh=mesh,
        compiler_params=pltpu.CompilerParams(use_tc_tiling_on_sc=False, skip_device_barrier=True),
        out_type=jax.ShapeDtypeStruct((FLAT // 2, 2 * D), jnp.float32),
        scratch_types=[
            pltpu.VMEM((2, C), jnp.int32),
            pltpu.VMEM((2, C, D), jnp.float32),
            pltpu.VMEM((2, C2, 2 * D), jnp.float32),
            pltpu.VMEM((L, 2 * D), jnp.float32),
            pltpu.SemaphoreType.DMA,
            pltpu.SemaphoreType.DMA,
            pltpu.SemaphoreType.DMA,
            pltpu.SemaphoreType.DMA,
            pltpu.SemaphoreType.DMA,
            pltpu.SemaphoreType.DMA,
        ],
    )
    def lookup(table_hbm, idx_hbm, pe_hbm, out_hbm, idx_v, rin_v, rout_v,
               pe_v, g0, g1, i0, i1, s0, s1):
        gsem = (g0, g1)
        isem = (i0, i1)
        ssem = (s0, s1)
        wid = lax.axis_index("s") * NC + lax.axis_index("c")
        gbase = wid * NCHUNK
        obase = wid * (PER_W // 2)

        pltpu.sync_copy(pe_hbm, pe_v)
        pltpu.sync_copy(idx_hbm.at[gbase], idx_v.at[0])
        pltpu.async_copy(idx_hbm.at[gbase + 1], idx_v.at[1], isem[1])
        pltpu.async_copy(table_hbm.at[idx_v.at[0]], rin_v.at[0], gsem[0])

        @pl.loop(0, NCHUNK, step=2)
        def chunks(cc):
            for b in range(2):
                c = cc + b
                pltpu.make_async_copy(
                    table_hbm.at[idx_v.at[b]], rin_v.at[b], gsem[b]).wait()

                @pl.when(c + 2 < NCHUNK)
                def _():
                    pltpu.async_copy(
                        idx_hbm.at[gbase + c + 2], idx_v.at[b], isem[b])

                @pl.when(c + 1 < NCHUNK)
                def _():
                    pltpu.make_async_copy(
                        idx_hbm.at[gbase + c + 1], idx_v.at[1 - b],
                        isem[1 - b]).wait()
                    pltpu.async_copy(
                        table_hbm.at[idx_v.at[1 - b]], rin_v.at[1 - b],
                        gsem[1 - b])

                @pl.when(c >= 2)
                def _():
                    pltpu.make_async_copy(
                        rout_v.at[b],
                        out_hbm.at[pl.ds(obase + (c - 2) * C2, C2)],
                        ssem[b]).wait()

                rin = rin_v.at[b]
                rout = rout_v.at[b]
                pp = lax.rem(c * C2, L // 2)

                @plsc.parallel_loop(0, C2, unroll=2)
                def row(r2):
                    pr = pp + r2
                    for d in range(2 * D // NL):
                        s_ = pl.ds(d * NL, NL)
                        rout[r2, s_] = (
                            rin[2 * r2 + d // (D // NL),
                                pl.ds((d % (D // NL)) * NL, NL)] * scale2
                            + pe_v[pr, s_])

                pltpu.async_copy(rout_v.at[b],
                                 out_hbm.at[pl.ds(obase + c * C2, C2)],
                                 ssem[b])

        for b in range(2):
            c = NCHUNK - 2 + b
            pltpu.make_async_copy(
                rout_v.at[b], out_hbm.at[pl.ds(obase + c * C2, C2)],
                ssem[b]).wait()

    return lookup


def kernel(x, table, pe):
    B, L = x.shape
    V, D = table.shape
    FLAT = B * L
    x_rows = x.reshape(FLAT // 128, 128).astype(jnp.int32)
    pe2 = jnp.tile(pe[:L] * math.sqrt(D), (2, 1)).reshape(L, 2 * D)
    out = _make_sc_lookup(V, D, FLAT, L)(table, x_rows, pe2)
    return out.reshape(B, L, D)
